# baseline (speedup 1.0000x reference)
import jax
import jax.numpy as jnp
from jax.experimental import pallas as pl

S = 2048
DIM = 768
H = 12
DH = 64
BLK = 4
NB = S // BLK
WIN = 32
QC = 256
NQ = S // QC
GCOL = 128
SCALE = DH ** -0.5
F32 = jnp.float32
BF16 = jnp.bfloat16


def _prep_kernel(x_ref, x0_ref, lam_ref, w_ref, x1_ref, q_ref, k_ref, v_ref,
                 g_ref):
    lam0 = lam_ref[0, 0]
    lam1 = lam_ref[0, 1]
    x1 = lam0 * x_ref[...] + lam1 * x0_ref[...]
    x1_ref[...] = x1
    h = x1 * jax.lax.rsqrt(jnp.mean(x1 * x1, axis=-1, keepdims=True) + 1e-6)
    y = jnp.dot(h.astype(BF16), w_ref[...], preferred_element_type=F32)
    for hh in range(H):
        q_ref[hh] = (y[:, hh * DH : (hh + 1) * DH] * SCALE).astype(BF16)
        k_ref[hh] = y[:, DIM + hh * DH : DIM + (hh + 1) * DH].astype(BF16)
        v_ref[hh] = y[:, 2 * DIM + hh * DH : 2 * DIM + (hh + 1) * DH].astype(BF16)
        g_ref[hh] = jax.nn.sigmoid(y[:, 3 * DIM + 3 * hh : 3 * DIM + 3 * hh + 4])


def _ckv_kernel(kh_ref, vh_ref, wkc_ref, wvc_ref, kpe_ref, vpe_ref, ck_ref, cv_ref):
    pe_k = jnp.dot(kpe_ref[...], wkc_ref[...], preferred_element_type=F32)
    pe_v = jnp.dot(vpe_ref[...], wvc_ref[...], preferred_element_type=F32)
    ck_ref[0] = (jnp.dot(kh_ref[0], wkc_ref[...], preferred_element_type=F32) + pe_k).astype(BF16)
    cv_ref[0] = (jnp.dot(vh_ref[0], wvc_ref[...], preferred_element_type=F32) + pe_v).astype(BF16)


QR = 512
NCALL = S // QR
SB = QR + 64


def _attn_one_head(j, q, k, v, ck, cv, fmadd, cmask, smask, g):
    W = (j + 1) * QR
    s0 = max(0, j * QR - 64)
    sim = jax.lax.dot_general(q, k, (((1,), (1,)), ((), ())),
                              preferred_element_type=F32)

    sf = sim + fmadd
    mf = jnp.max(sf, axis=-1, keepdims=True)
    pf = jnp.exp(sf - mf)
    f_out = jax.lax.dot_general(pf.astype(BF16), v, (((1,), (0,)), ((), ())),
                                preferred_element_type=F32) / jnp.sum(
        pf, axis=-1, keepdims=True
    )

    simc = jax.lax.dot_general(q, ck, (((1,), (1,)), ((), ())),
                               preferred_element_type=F32)
    sc = simc + cmask
    mc = jnp.maximum(jnp.max(sc, axis=-1, keepdims=True), 0.0)
    pc = jnp.exp(sc - mc)
    den = jnp.sum(pc, axis=-1, keepdims=True) + jnp.exp(-mc)
    c_out = jax.lax.dot_general(pc.astype(BF16), cv, (((1,), (0,)), ((), ())),
                                preferred_element_type=F32) / den

    sbw = min(SB, W - s0)
    kb = k[s0 : s0 + sbw, :]
    vb = v[s0 : s0 + sbw, :]
    sims = jax.lax.dot_general(q, kb, (((1,), (1,)), ((), ())),
                               preferred_element_type=F32)
    ss = sims + smask
    ms = jnp.max(ss, axis=-1, keepdims=True)
    ps = jnp.exp(ss - ms)
    s_out = jax.lax.dot_general(ps.astype(BF16), vb, (((1,), (0,)), ((), ())),
                                preferred_element_type=F32) / jnp.sum(
        ps, axis=-1, keepdims=True
    )

    return g[:, 0:1] * c_out + g[:, 1:2] * f_out + g[:, 2:3] * s_out


def _attn_band_kernel(q_ref, k_ref, v_ref, ck_ref, cv_ref, fm0_ref, fm1_ref,
                      fm2_ref, fm3_ref, g_ref, _acc_ref, out_ref):
    j = pl.program_id(1)
    fm_refs = [fm0_ref, fm1_ref, fm2_ref, fm3_ref]

    def mk(jj):
        W = (jj + 1) * QR
        CB = W // BLK
        s0 = max(0, jj * QR - 64)
        sbw = min(SB, W - s0)

        def br():
            fmadd = (fm_refs[jj][...].astype(F32) - 1.0) * 1e9
            rowc = jj * QR + jax.lax.broadcasted_iota(jnp.int32, (QR, CB), 0)
            colc = jax.lax.broadcasted_iota(jnp.int32, (QR, CB), 1)
            cmask = jnp.where(((colc + 1) * BLK - 1) <= rowc, 0.0, -1e9)
            rows = jj * QR + jax.lax.broadcasted_iota(jnp.int32, (QR, sbw), 0)
            cols = s0 + jax.lax.broadcasted_iota(jnp.int32, (QR, sbw), 1)
            smask = jnp.where((cols <= rows) & ((rows - cols) < WIN), 0.0, -1e9)
            outs = []
            for t in range(2):
                outs.append(_attn_one_head(
                    jj, q_ref[t], k_ref[t, :W, :], v_ref[t, :W, :],
                    ck_ref[t, :CB, :], cv_ref[t, :CB, :],
                    fmadd, cmask, smask, g_ref[t]))
            out_ref[...] = jnp.concatenate(outs, axis=-1).astype(BF16)
        return br

    jax.lax.switch(j, [mk(0), mk(1), mk(2), mk(3)])


def _mlp_kernel(attn_ref, x1_ref, wo_ref, wfc_ref, wproj_ref, y_ref):
    x2 = x1_ref[...] + jnp.dot(attn_ref[...], wo_ref[...], preferred_element_type=F32)
    h2 = x2 * jax.lax.rsqrt(jnp.mean(x2 * x2, axis=-1, keepdims=True) + 1e-6)
    u = jnp.dot(h2.astype(BF16), wfc_ref[...], preferred_element_type=F32)
    u = jnp.square(jnp.maximum(u, 0.0))
    y_ref[...] = x2 + jnp.dot(u.astype(BF16), wproj_ref[...], preferred_element_type=F32)


def kernel(x, ve, x0, lambdas, Wq, Wk, Wv, Wo, k_pe, v_pe, Wkc, Wvc, Wg, Wfc, Wproj,
           sliding_window_flex_mask, fine_selection_flex_mask):
    del ve, sliding_window_flex_mask
    x2d = x[0]
    x02d = x0[0]
    w_all = jnp.concatenate(
        [Wq, Wk, Wv, jnp.pad(Wg, ((0, 0), (0, GCOL - 3 * H)))], axis=1
    ).astype(BF16)
    lam2 = lambdas.reshape(1, 2)

    x1, q_hm, k_hm, v_hm, g_hm = pl.pallas_call(
        _prep_kernel,
        grid=(NQ,),
        in_specs=[
            pl.BlockSpec((QC, DIM), lambda i: (i, 0)),
            pl.BlockSpec((QC, DIM), lambda i: (i, 0)),
            pl.BlockSpec((1, 2), lambda i: (0, 0)),
            pl.BlockSpec((DIM, 3 * DIM + GCOL), lambda i: (0, 0)),
        ],
        out_specs=[
            pl.BlockSpec((QC, DIM), lambda i: (i, 0)),
            pl.BlockSpec((H, QC, DH), lambda i: (0, i, 0)),
            pl.BlockSpec((H, QC, DH), lambda i: (0, i, 0)),
            pl.BlockSpec((H, QC, DH), lambda i: (0, i, 0)),
            pl.BlockSpec((H, QC, 4), lambda i: (0, i, 0)),
        ],
        out_shape=[
            jax.ShapeDtypeStruct((S, DIM), F32),
            jax.ShapeDtypeStruct((H, S, DH), BF16),
            jax.ShapeDtypeStruct((H, S, DH), BF16),
            jax.ShapeDtypeStruct((H, S, DH), BF16),
            jax.ShapeDtypeStruct((H, S, 4), F32),
        ],
    )(x2d, x02d, lam2, w_all)

    kh = k_hm.reshape(H, NB, BLK * DH)
    vh = v_hm.reshape(H, NB, BLK * DH)

    ck, cv = pl.pallas_call(
        _ckv_kernel,
        grid=(H,),
        in_specs=[
            pl.BlockSpec((1, NB, BLK * DH), lambda h: (h, 0, 0)),
            pl.BlockSpec((1, NB, BLK * DH), lambda h: (h, 0, 0)),
            pl.BlockSpec((BLK * DH, DH), lambda h: (0, 0)),
            pl.BlockSpec((BLK * DH, DH), lambda h: (0, 0)),
            pl.BlockSpec((1, BLK * DH), lambda h: (0, 0)),
            pl.BlockSpec((1, BLK * DH), lambda h: (0, 0)),
        ],
        out_specs=[
            pl.BlockSpec((1, NB, DH), lambda h: (h, 0, 0)),
            pl.BlockSpec((1, NB, DH), lambda h: (h, 0, 0)),
        ],
        out_shape=[
            jax.ShapeDtypeStruct((H, NB, DH), BF16),
            jax.ShapeDtypeStruct((H, NB, DH), BF16),
        ],
    )(kh, vh, Wkc.astype(BF16), Wvc.astype(BF16),
      k_pe.reshape(1, BLK * DH).astype(BF16), v_pe.reshape(1, BLK * DH).astype(BF16))

    fm8 = fine_selection_flex_mask.astype(jnp.int8)

    fms = [fm8[j * QR : (j + 1) * QR, : (j + 1) * QR] for j in range(NCALL)]

    attn2d = pl.pallas_call(
        _attn_band_kernel,
        grid=(H // 2, NCALL),
        in_specs=[
            pl.BlockSpec((2, QR, DH), lambda m, j: (m, j, 0)),
            pl.BlockSpec((2, S, DH), lambda m, j: (m, 0, 0)),
            pl.BlockSpec((2, S, DH), lambda m, j: (m, 0, 0)),
            pl.BlockSpec((2, NB, DH), lambda m, j: (m, 0, 0)),
            pl.BlockSpec((2, NB, DH), lambda m, j: (m, 0, 0)),
            pl.BlockSpec((QR, QR), lambda m, j: (0, 0)),
            pl.BlockSpec((QR, 2 * QR), lambda m, j: (0, 0)),
            pl.BlockSpec((QR, 3 * QR), lambda m, j: (0, 0)),
            pl.BlockSpec((QR, 4 * QR), lambda m, j: (0, 0)),
            pl.BlockSpec((2, QR, 4), lambda m, j: (m, j, 0)),
            pl.BlockSpec((QR, 2 * DH), lambda m, j: (j, m)),
        ],
        out_specs=pl.BlockSpec((QR, 2 * DH), lambda m, j: (j, m)),
        out_shape=jax.ShapeDtypeStruct((S, H * DH), BF16),
        input_output_aliases={10: 0},
    )(q_hm, k_hm, v_hm, ck, cv, fms[0], fms[1], fms[2], fms[3], g_hm,
      jnp.zeros((S, H * DH), BF16))

    out = pl.pallas_call(
        _mlp_kernel,
        grid=(NQ,),
        in_specs=[
            pl.BlockSpec((QC, H * DH), lambda i: (i, 0)),
            pl.BlockSpec((QC, DIM), lambda i: (i, 0)),
            pl.BlockSpec((H * DH, DIM), lambda i: (0, 0)),
            pl.BlockSpec((DIM, 4 * DIM), lambda i: (0, 0)),
            pl.BlockSpec((4 * DIM, DIM), lambda i: (0, 0)),
        ],
        out_specs=pl.BlockSpec((QC, DIM), lambda i: (i, 0)),
        out_shape=jax.ShapeDtypeStruct((S, DIM), F32),
    )(attn2d, x1, Wo.astype(BF16), Wfc.astype(BF16), Wproj.astype(BF16))

    return out[None]


# speedup vs baseline: 1.0175x; 1.0175x over previous
import jax
import jax.numpy as jnp
from jax.experimental import pallas as pl

S = 2048
DIM = 768
H = 12
DH = 64
BLK = 4
NB = S // BLK
WIN = 32
QC = 256
NQ = S // QC
GCOL = 128
SCALE = DH ** -0.5
F32 = jnp.float32
BF16 = jnp.bfloat16


def _prep_kernel(x_ref, x0_ref, lam_ref, w_ref, x1_ref, q_ref, k_ref, v_ref,
                 g_ref):
    lam0 = lam_ref[0, 0]
    lam1 = lam_ref[0, 1]
    x1 = lam0 * x_ref[...] + lam1 * x0_ref[...]
    x1_ref[...] = x1
    h = x1 * jax.lax.rsqrt(jnp.mean(x1 * x1, axis=-1, keepdims=True) + 1e-6)
    y = jnp.dot(h.astype(BF16), w_ref[...], preferred_element_type=F32)
    for hh in range(H):
        q_ref[hh] = (y[:, hh * DH : (hh + 1) * DH] * SCALE).astype(BF16)
        k_ref[hh] = y[:, DIM + hh * DH : DIM + (hh + 1) * DH].astype(BF16)
        v_ref[hh] = y[:, 2 * DIM + hh * DH : 2 * DIM + (hh + 1) * DH].astype(BF16)
        g_ref[hh] = jax.nn.sigmoid(y[:, 3 * DIM + 3 * hh : 3 * DIM + 3 * hh + 4])


def _ckv_kernel(kh_ref, vh_ref, wkc_ref, wvc_ref, kpe_ref, vpe_ref, ck_ref, cv_ref):
    pe_k = jnp.dot(kpe_ref[...], wkc_ref[...], preferred_element_type=F32)
    pe_v = jnp.dot(vpe_ref[...], wvc_ref[...], preferred_element_type=F32)
    ck_ref[0] = (jnp.dot(kh_ref[0], wkc_ref[...], preferred_element_type=F32) + pe_k).astype(BF16)
    cv_ref[0] = (jnp.dot(vh_ref[0], wvc_ref[...], preferred_element_type=F32) + pe_v).astype(BF16)


QR = 512
NCALL = S // QR
SB = QR + 64


def _attn_one_head(j, q, k, v, ck, cv, fmadd, cmask, smask, g):
    W = (j + 1) * QR
    s0 = max(0, j * QR - 64)
    sim = jax.lax.dot_general(q, k, (((1,), (1,)), ((), ())),
                              preferred_element_type=F32)

    sf = sim + fmadd
    mf = jnp.max(sf, axis=-1, keepdims=True)
    pf = jnp.exp(sf - mf)
    f_out = jax.lax.dot_general(pf.astype(BF16), v, (((1,), (0,)), ((), ())),
                                preferred_element_type=F32) / jnp.sum(
        pf, axis=-1, keepdims=True
    )

    simc = jax.lax.dot_general(q, ck, (((1,), (1,)), ((), ())),
                               preferred_element_type=F32)
    sc = simc + cmask
    mc = jnp.maximum(jnp.max(sc, axis=-1, keepdims=True), 0.0)
    pc = jnp.exp(sc - mc)
    den = jnp.sum(pc, axis=-1, keepdims=True) + jnp.exp(-mc)
    c_out = jax.lax.dot_general(pc.astype(BF16), cv, (((1,), (0,)), ((), ())),
                                preferred_element_type=F32) / den

    sbw = min(SB, W - s0)
    kb = k[s0 : s0 + sbw, :]
    vb = v[s0 : s0 + sbw, :]
    sims = jax.lax.dot_general(q, kb, (((1,), (1,)), ((), ())),
                               preferred_element_type=F32)
    ss = sims + smask
    ms = jnp.max(ss, axis=-1, keepdims=True)
    ps = jnp.exp(ss - ms)
    s_out = jax.lax.dot_general(ps.astype(BF16), vb, (((1,), (0,)), ((), ())),
                                preferred_element_type=F32) / jnp.sum(
        ps, axis=-1, keepdims=True
    )

    return g[:, 0:1] * c_out + g[:, 1:2] * f_out + g[:, 2:3] * s_out


def _attn_band_kernel(q_ref, k_ref, v_ref, ck_ref, cv_ref, fm_ref, g_ref,
                      _acc_ref, out_ref):
    j = pl.program_id(0)

    def mk(jj):
        W = (jj + 1) * QR
        CB = W // BLK
        s0 = max(0, jj * QR - 64)
        sbw = min(SB, W - s0)

        def br():
            fmadd = (fm_ref[:, :W].astype(F32) - 1.0) * 1e9
            rowc = jj * QR + jax.lax.broadcasted_iota(jnp.int32, (QR, CB), 0)
            colc = jax.lax.broadcasted_iota(jnp.int32, (QR, CB), 1)
            cmask = jnp.where(((colc + 1) * BLK - 1) <= rowc, 0.0, -1e9)
            rows = jj * QR + jax.lax.broadcasted_iota(jnp.int32, (QR, sbw), 0)
            cols = s0 + jax.lax.broadcasted_iota(jnp.int32, (QR, sbw), 1)
            smask = jnp.where((cols <= rows) & ((rows - cols) < WIN), 0.0, -1e9)
            outs = []
            for t in range(2):
                outs.append(_attn_one_head(
                    jj, q_ref[t], k_ref[t, :W, :], v_ref[t, :W, :],
                    ck_ref[t, :CB, :], cv_ref[t, :CB, :],
                    fmadd, cmask, smask, g_ref[t]))
            out_ref[...] = jnp.concatenate(outs, axis=-1).astype(BF16)
        return br

    jax.lax.switch(j, [mk(0), mk(1), mk(2), mk(3)])


def _mlp_kernel(attn_ref, x1_ref, wo_ref, wfc_ref, wproj_ref, y_ref):
    x2 = x1_ref[...] + jnp.dot(attn_ref[...], wo_ref[...], preferred_element_type=F32)
    h2 = x2 * jax.lax.rsqrt(jnp.mean(x2 * x2, axis=-1, keepdims=True) + 1e-6)
    u = jnp.dot(h2.astype(BF16), wfc_ref[...], preferred_element_type=F32)
    u = jnp.square(jnp.maximum(u, 0.0))
    y_ref[...] = x2 + jnp.dot(u.astype(BF16), wproj_ref[...], preferred_element_type=F32)


def kernel(x, ve, x0, lambdas, Wq, Wk, Wv, Wo, k_pe, v_pe, Wkc, Wvc, Wg, Wfc, Wproj,
           sliding_window_flex_mask, fine_selection_flex_mask):
    del ve, sliding_window_flex_mask
    x2d = x[0]
    x02d = x0[0]
    w_all = jnp.concatenate(
        [Wq, Wk, Wv, jnp.pad(Wg, ((0, 0), (0, GCOL - 3 * H)))], axis=1
    ).astype(BF16)
    lam2 = lambdas.reshape(1, 2)

    x1, q_hm, k_hm, v_hm, g_hm = pl.pallas_call(
        _prep_kernel,
        grid=(NQ,),
        in_specs=[
            pl.BlockSpec((QC, DIM), lambda i: (i, 0)),
            pl.BlockSpec((QC, DIM), lambda i: (i, 0)),
            pl.BlockSpec((1, 2), lambda i: (0, 0)),
            pl.BlockSpec((DIM, 3 * DIM + GCOL), lambda i: (0, 0)),
        ],
        out_specs=[
            pl.BlockSpec((QC, DIM), lambda i: (i, 0)),
            pl.BlockSpec((H, QC, DH), lambda i: (0, i, 0)),
            pl.BlockSpec((H, QC, DH), lambda i: (0, i, 0)),
            pl.BlockSpec((H, QC, DH), lambda i: (0, i, 0)),
            pl.BlockSpec((H, QC, 4), lambda i: (0, i, 0)),
        ],
        out_shape=[
            jax.ShapeDtypeStruct((S, DIM), F32),
            jax.ShapeDtypeStruct((H, S, DH), BF16),
            jax.ShapeDtypeStruct((H, S, DH), BF16),
            jax.ShapeDtypeStruct((H, S, DH), BF16),
            jax.ShapeDtypeStruct((H, S, 4), F32),
        ],
    )(x2d, x02d, lam2, w_all)

    kh = k_hm.reshape(H, NB, BLK * DH)
    vh = v_hm.reshape(H, NB, BLK * DH)

    ck, cv = pl.pallas_call(
        _ckv_kernel,
        grid=(H,),
        in_specs=[
            pl.BlockSpec((1, NB, BLK * DH), lambda h: (h, 0, 0)),
            pl.BlockSpec((1, NB, BLK * DH), lambda h: (h, 0, 0)),
            pl.BlockSpec((BLK * DH, DH), lambda h: (0, 0)),
            pl.BlockSpec((BLK * DH, DH), lambda h: (0, 0)),
            pl.BlockSpec((1, BLK * DH), lambda h: (0, 0)),
            pl.BlockSpec((1, BLK * DH), lambda h: (0, 0)),
        ],
        out_specs=[
            pl.BlockSpec((1, NB, DH), lambda h: (h, 0, 0)),
            pl.BlockSpec((1, NB, DH), lambda h: (h, 0, 0)),
        ],
        out_shape=[
            jax.ShapeDtypeStruct((H, NB, DH), BF16),
            jax.ShapeDtypeStruct((H, NB, DH), BF16),
        ],
    )(kh, vh, Wkc.astype(BF16), Wvc.astype(BF16),
      k_pe.reshape(1, BLK * DH).astype(BF16), v_pe.reshape(1, BLK * DH).astype(BF16))

    fm8 = fine_selection_flex_mask.astype(jnp.int8)

    attn2d = pl.pallas_call(
        _attn_band_kernel,
        grid=(NCALL, H // 2),
        in_specs=[
            pl.BlockSpec((2, QR, DH), lambda j, m: (m, j, 0)),
            pl.BlockSpec((2, S, DH), lambda j, m: (m, 0, 0)),
            pl.BlockSpec((2, S, DH), lambda j, m: (m, 0, 0)),
            pl.BlockSpec((2, NB, DH), lambda j, m: (m, 0, 0)),
            pl.BlockSpec((2, NB, DH), lambda j, m: (m, 0, 0)),
            pl.BlockSpec((QR, S), lambda j, m: (j, 0)),
            pl.BlockSpec((2, QR, 4), lambda j, m: (m, j, 0)),
            pl.BlockSpec((QR, 2 * DH), lambda j, m: (j, m)),
        ],
        out_specs=pl.BlockSpec((QR, 2 * DH), lambda j, m: (j, m)),
        out_shape=jax.ShapeDtypeStruct((S, H * DH), BF16),
        input_output_aliases={7: 0},
    )(q_hm, k_hm, v_hm, ck, cv, fm8, g_hm, jnp.zeros((S, H * DH), BF16))

    out = pl.pallas_call(
        _mlp_kernel,
        grid=(NQ,),
        in_specs=[
            pl.BlockSpec((QC, H * DH), lambda i: (i, 0)),
            pl.BlockSpec((QC, DIM), lambda i: (i, 0)),
            pl.BlockSpec((H * DH, DIM), lambda i: (0, 0)),
            pl.BlockSpec((DIM, 4 * DIM), lambda i: (0, 0)),
            pl.BlockSpec((4 * DIM, DIM), lambda i: (0, 0)),
        ],
        out_specs=pl.BlockSpec((QC, DIM), lambda i: (i, 0)),
        out_shape=jax.ShapeDtypeStruct((S, DIM), F32),
    )(attn2d, x1, Wo.astype(BF16), Wfc.astype(BF16), Wproj.astype(BF16))

    return out[None]
